# baseline (speedup 1.0000x reference)
import functools

import jax
import jax.numpy as jnp
from jax.experimental import pallas as pl
from jax.experimental.pallas import tpu as pltpu


def _se_fused_kernel(x_ref, w_ref, o_ref, *, hw, inv_hw):
    x = x_ref[...]
    lane = jax.lax.broadcasted_iota(jnp.int32, (1, 2 * hw), 1)
    m = jnp.where(lane < hw, jnp.float32(1.0), jnp.float32(0.0))

    xm = x * m
    s_even = jnp.sum(xm, axis=-1, keepdims=True, dtype=jnp.float32)
    s_total = jnp.sum(x, axis=-1, keepdims=True, dtype=jnp.float32)
    s_odd = s_total - s_even

    mean_vec = jnp.concatenate([s_even, s_odd], axis=0) * inv_hw
    z = jnp.dot(w_ref[...], mean_vec, preferred_element_type=jnp.float32)
    gate = jax.nn.sigmoid(z)
    ge = gate[: x.shape[0], :].astype(x.dtype)
    go = gate[x.shape[0] :, :].astype(x.dtype)

    o_ref[...] = x * go + xm * (ge - go)


def kernel(x_nchw, weight):
    N, C, H, W = x_nchw.shape
    HW = H * W
    C2 = C // 2
    x = x_nchw.reshape(N, C2, 2 * HW)

    w = weight.astype(jnp.float32)
    w_cols = jnp.concatenate([w[:, 0::2], w[:, 1::2]], axis=1)
    w_p = jnp.concatenate([w_cols[0::2, :], w_cols[1::2, :]], axis=0)

    body = functools.partial(_se_fused_kernel, hw=HW, inv_hw=float(1.0 / HW))

    itemsize = jnp.dtype(x.dtype).itemsize
    cost = pl.CostEstimate(
        flops=5 * N * C * HW + 2 * N * C * C,
        transcendentals=N * C,
        bytes_accessed=2 * N * C * HW * itemsize + C * C * 4,
    )
    out = pl.pallas_call(
        body,
        out_shape=jax.ShapeDtypeStruct((N, C2, 2 * HW), x.dtype),
        grid=(N,),
        in_specs=[
            pl.BlockSpec((pl.Squeezed(), C2, 2 * HW), lambda n: (n, 0, 0)),
            pl.BlockSpec((C, C), lambda n: (0, 0)),
        ],
        out_specs=pl.BlockSpec((pl.Squeezed(), C2, 2 * HW), lambda n: (n, 0, 0)),
        compiler_params=pltpu.CompilerParams(
            dimension_semantics=("parallel",),
            vmem_limit_bytes=64 * 1024 * 1024,
        ),
        cost_estimate=cost,
    )(x, w_p)
    return out.reshape(N, C, H, W)


# speedup vs baseline: 9.4997x; 9.4997x over previous
import functools

import jax
import jax.numpy as jnp
from jax.experimental import pallas as pl
from jax.experimental.pallas import tpu as pltpu


def _se_fused_kernel(x_ref, wt_ref, o_ref, *, inv_hw):
    x = x_ref[...]
    mean = jnp.sum(x, axis=0, keepdims=True, dtype=jnp.float32) * inv_hw
    z = jnp.dot(mean, wt_ref[...], preferred_element_type=jnp.float32)
    gate = jax.nn.sigmoid(z).astype(x.dtype)
    o_ref[...] = x * gate


def kernel(x_nchw, weight):
    N, C, H, W = x_nchw.shape
    HW = H * W
    x = jnp.transpose(x_nchw, (0, 2, 3, 1)).reshape(N, HW, C)
    w_t = weight.astype(jnp.float32).T

    body = functools.partial(_se_fused_kernel, inv_hw=float(1.0 / HW))

    itemsize = jnp.dtype(x.dtype).itemsize
    cost = pl.CostEstimate(
        flops=3 * N * C * HW + 2 * N * C * C,
        transcendentals=N * C,
        bytes_accessed=2 * N * C * HW * itemsize + C * C * 4,
    )
    out = pl.pallas_call(
        body,
        out_shape=jax.ShapeDtypeStruct((N, HW, C), x.dtype),
        grid=(N,),
        in_specs=[
            pl.BlockSpec((pl.Squeezed(), HW, C), lambda n: (n, 0, 0)),
            pl.BlockSpec((C, C), lambda n: (0, 0)),
        ],
        out_specs=pl.BlockSpec((pl.Squeezed(), HW, C), lambda n: (n, 0, 0)),
        compiler_params=pltpu.CompilerParams(
            dimension_semantics=("parallel",),
            vmem_limit_bytes=64 * 1024 * 1024,
        ),
        cost_estimate=cost,
    )(x, w_t)
    return jnp.transpose(out.reshape(N, H, W, C), (0, 3, 1, 2))


# speedup vs baseline: 10.0668x; 1.0597x over previous
import functools

import jax
import jax.numpy as jnp
from jax.experimental import pallas as pl
from jax.experimental.pallas import tpu as pltpu


def _se_fused_kernel(x_ref, wt_ref, o_ref, *, inv_hw):
    x = x_ref[...]
    mean = jnp.sum(x, axis=1, dtype=jnp.float32) * inv_hw
    z = jnp.dot(mean, wt_ref[...], preferred_element_type=jnp.float32)
    gate = jax.nn.sigmoid(z).astype(x.dtype)
    o_ref[...] = x * gate[:, None, :]


def kernel(x_nchw, weight):
    N, C, H, W = x_nchw.shape
    HW = H * W
    x = jnp.transpose(x_nchw, (0, 2, 3, 1)).reshape(N, HW, C)
    w_t = weight.astype(jnp.float32).T

    body = functools.partial(_se_fused_kernel, inv_hw=float(1.0 / HW))

    itemsize = jnp.dtype(x.dtype).itemsize
    cost = pl.CostEstimate(
        flops=3 * N * C * HW + 2 * N * C * C,
        transcendentals=N * C,
        bytes_accessed=2 * N * C * HW * itemsize + C * C * 4,
    )
    B = 2
    out = pl.pallas_call(
        body,
        out_shape=jax.ShapeDtypeStruct((N, HW, C), x.dtype),
        grid=(N // B,),
        in_specs=[
            pl.BlockSpec((B, HW, C), lambda n: (n, 0, 0)),
            pl.BlockSpec((C, C), lambda n: (0, 0)),
        ],
        out_specs=pl.BlockSpec((B, HW, C), lambda n: (n, 0, 0)),
        compiler_params=pltpu.CompilerParams(
            dimension_semantics=("parallel",),
            vmem_limit_bytes=64 * 1024 * 1024,
        ),
        cost_estimate=cost,
    )(x, w_t)
    return jnp.transpose(out.reshape(N, H, W, C), (0, 3, 1, 2))


# speedup vs baseline: 10.2103x; 1.0142x over previous
import functools

import jax
import jax.numpy as jnp
from jax.experimental import pallas as pl
from jax.experimental.pallas import tpu as pltpu


def _se_fused_kernel(x_ref, wt_ref, o_ref, *, inv_hw):
    x = x_ref[...]
    mean = jnp.sum(x, axis=1, dtype=jnp.float32) * inv_hw
    z = jnp.dot(mean, wt_ref[...], preferred_element_type=jnp.float32)
    gate = jax.nn.sigmoid(z).astype(x.dtype)
    o_ref[...] = x * gate[:, None, :]


def kernel(x_nchw, weight):
    N, C, H, W = x_nchw.shape
    HW = H * W
    x = jnp.transpose(x_nchw, (0, 2, 3, 1)).reshape(N, HW, C)
    w_t = weight.astype(jnp.float32).T

    body = functools.partial(_se_fused_kernel, inv_hw=float(1.0 / HW))

    itemsize = jnp.dtype(x.dtype).itemsize
    cost = pl.CostEstimate(
        flops=3 * N * C * HW + 2 * N * C * C,
        transcendentals=N * C,
        bytes_accessed=2 * N * C * HW * itemsize + C * C * 4,
    )
    B = 4
    out = pl.pallas_call(
        body,
        out_shape=jax.ShapeDtypeStruct((N, HW, C), x.dtype),
        grid=(N // B,),
        in_specs=[
            pl.BlockSpec((B, HW, C), lambda n: (n, 0, 0)),
            pl.BlockSpec((C, C), lambda n: (0, 0)),
        ],
        out_specs=pl.BlockSpec((B, HW, C), lambda n: (n, 0, 0)),
        compiler_params=pltpu.CompilerParams(
            dimension_semantics=("parallel",),
            vmem_limit_bytes=64 * 1024 * 1024,
        ),
        cost_estimate=cost,
    )(x, w_t)
    return jnp.transpose(out.reshape(N, H, W, C), (0, 3, 1, 2))
